# Initial kernel scaffold; baseline (speedup 1.0000x reference)
import jax
import jax.numpy as jnp
from jax.experimental import pallas as pl


def kernel(atom_weights, orig_atom_fea, nbr_fea, self_fea_idx, nbr_fea_idx, crystal_atom_idx, W_emb, b_emb, W_g1, b_g1, g_gamma, g_beta, W_g2, b_g2, W_fc, b_fc, fc_gamma, fc_beta, W_out, b_out):
    raise NotImplementedError("write your pallas kernel here")


# speedup vs baseline: 5.3727x; 5.3727x over previous
import functools

import jax
import jax.numpy as jnp
from jax.experimental import pallas as pl
from jax.experimental.pallas import tpu as pltpu

N = 10000
C = 500
ORIG = 128
ATOM = 48
HID = 16
H = 128

BLK = 1024
NP = 10240
NBLK = NP // BLK
CP = 512

_EPS_BN = 1e-5
_EPS_DEN = 1e-13


def _body(x_ref, aw_ref, idxc_ref, idxr_ref,
          wemb_ref, bemb_ref, wg1_ref, bg1_ref, ggam_ref, gbet_ref,
          wg2_ref, bg2_ref, wfc_ref, bfc_ref, fgam_ref, fbet_ref,
          wout_ref, bout_ref, out_ref, af_ref):
    f32 = jnp.float32
    wemb = wemb_ref[:, :]
    bemb = bemb_ref[:, :]
    wg1 = wg1_ref[:, :]
    bg1 = bg1_ref[:, :]
    wg2 = wg2_ref[:, :]
    bg2 = bg2_ref[:, :]

    def loop_a(t, carry):
        s1, s2 = carry
        xb = x_ref[pl.ds(t * BLK, BLK), :]
        af = jnp.dot(xb, wemb, preferred_element_type=f32) + bemb
        af_ref[pl.ds(t * BLK, BLK), :] = af
        z = jnp.dot(af, wg1, preferred_element_type=f32) + bg1
        rmask = (jax.lax.broadcasted_iota(jnp.int32, (BLK, 1), 0)
                 + t * BLK < N).astype(f32)
        zm = z * rmask
        s1 = s1 + jnp.sum(zm, axis=0, keepdims=True)
        s2 = s2 + jnp.sum(zm * z, axis=0, keepdims=True)
        return s1, s2

    s1, s2 = jax.lax.fori_loop(
        0, NBLK, loop_a,
        (jnp.zeros((1, HID), f32), jnp.zeros((1, HID), f32)))
    mean = s1 / N
    var = s2 / N - mean * mean
    scale = ggam_ref[:, :] * jax.lax.rsqrt(var + _EPS_BN)
    shift = gbet_ref[:, :] - mean * scale

    def gate_block(t):
        af = af_ref[pl.ds(t * BLK, BLK), :]
        z = jnp.dot(af, wg1, preferred_element_type=f32) + bg1
        h = jnp.maximum(z * scale + shift, 0.0)
        g = jnp.dot(h, wg2, preferred_element_type=f32) + bg2
        return af, g

    ciota_row = jax.lax.broadcasted_iota(jnp.int32, (1, CP), 1)
    ciota_col = jax.lax.broadcasted_iota(jnp.int32, (CP, 1), 0)
    neg_inf = jnp.float32(-jnp.inf)

    def loop_b(t, smax):
        _, g = gate_block(t)
        oh = idxc_ref[pl.ds(t * BLK, BLK), :] == ciota_row
        vals = jnp.where(oh, g, neg_inf)
        return jnp.maximum(smax, jnp.max(vals, axis=0, keepdims=True))

    smax = jax.lax.fori_loop(0, NBLK, loop_b,
                             jnp.full((1, CP), neg_inf, f32))

    def loop_c(t, carry):
        accf, accd = carry
        af, g = gate_block(t)
        oh = idxc_ref[pl.ds(t * BLK, BLK), :] == ciota_row
        gathered = jnp.max(jnp.where(oh, smax, neg_inf), axis=1,
                           keepdims=True)
        e = aw_ref[pl.ds(t * BLK, BLK), :] * jnp.exp(g - gathered)
        idxr = idxr_ref[t]
        oht = (ciota_col == idxr).astype(f32)
        accf = accf + jnp.dot(oht, af * e, preferred_element_type=f32)
        accd = accd + jnp.dot(oht, e, preferred_element_type=f32)
        return accf, accd

    accf, accd = jax.lax.fori_loop(
        0, NBLK, loop_c,
        (jnp.zeros((CP, ATOM), f32), jnp.zeros((CP, 1), f32)))
    crys = accf / (accd + _EPS_DEN)

    y = jnp.dot(crys, wfc_ref[:, :], preferred_element_type=f32) + bfc_ref[:, :]
    cmask = (ciota_col < C).astype(f32)
    ym = y * cmask
    m2 = jnp.sum(ym, axis=0, keepdims=True) / C
    v2 = jnp.sum(ym * y, axis=0, keepdims=True) / C - m2 * m2
    yn = (y - m2) * (fgam_ref[:, :] * jax.lax.rsqrt(v2 + _EPS_BN)) + fbet_ref[:, :]
    sp = jnp.maximum(yn, 0.0) + jnp.log1p(jnp.exp(-jnp.abs(yn)))
    out_ref[:, :] = jnp.dot(sp, wout_ref[:, :], preferred_element_type=f32) + bout_ref[:, :]


@jax.jit
def kernel(atom_weights, orig_atom_fea, nbr_fea, self_fea_idx, nbr_fea_idx,
           crystal_atom_idx, W_emb, b_emb, W_g1, b_g1, g_gamma, g_beta,
           W_g2, b_g2, W_fc, b_fc, fc_gamma, fc_beta, W_out, b_out):
    del nbr_fea, self_fea_idx, nbr_fea_idx
    f32 = jnp.float32
    pad = NP - N
    xp = jnp.pad(orig_atom_fea, ((0, pad), (0, 0)))
    awp = jnp.pad(atom_weights, ((0, pad), (0, 0)))
    idx = crystal_atom_idx.astype(jnp.int32)
    idxp = jnp.pad(idx, (0, pad), constant_values=CP - 1)
    idx_col = idxp.reshape(NP, 1)
    idx_rows = idxp.reshape(NBLK, 1, BLK)

    out = pl.pallas_call(
        _body,
        out_shape=jax.ShapeDtypeStruct((CP, 1), f32),
        scratch_shapes=[pltpu.VMEM((NP, ATOM), f32)],
    )(xp, awp, idx_col, idx_rows,
      W_emb, b_emb.reshape(1, ATOM), W_g1, b_g1.reshape(1, HID),
      g_gamma.reshape(1, HID), g_beta.reshape(1, HID),
      W_g2, b_g2.reshape(1, 1), W_fc, b_fc.reshape(1, H),
      fc_gamma.reshape(1, H), fc_beta.reshape(1, H),
      W_out, b_out.reshape(1, 1))
    return out[:C]
